# baseline (speedup 1.0000x reference)
import functools

import jax
import jax.numpy as jnp
from jax import lax
from jax.experimental import pallas as pl
from jax.experimental.pallas import tpu as pltpu
from jax.experimental.pallas import tpu_sc as plsc

VOCAB = 100000
EMB = 50
EMB_PAD = 128
NUM_F = 128
B = 16384
H = 64

_NC = 2
_NS = 16
_NW = _NC * _NS
_CHUNK = 128
_CPW = B // _NW // _CHUNK
_BPW = B // _NW


def _gather_body(idx_hbm, table_hbm, out_hbm, idx_v, rows_v, sem):
    wid = lax.axis_index("s") * _NC + lax.axis_index("c")
    pltpu.sync_copy(idx_hbm.at[pl.ds(wid * _CPW, _CPW)], idx_v)
    descs = []
    for j in range(_CPW):
        descs.append(
            pltpu.async_copy(
                table_hbm.at[idx_v.at[j]],
                rows_v.at[pl.ds(j * _CHUNK, _CHUNK)],
                sem,
            )
        )
    for d in descs:
        d.wait()
    pltpu.sync_copy(rows_v, out_hbm.at[pl.ds(wid * _BPW, _BPW)])


@functools.cache
def _gather():
    return pl.kernel(
        _gather_body,
        out_type=jax.ShapeDtypeStruct((B, EMB_PAD), jnp.float32),
        scratch_types=[
            pltpu.VMEM((_CPW, _CHUNK), jnp.int32),
            pltpu.VMEM((_BPW, EMB_PAD), jnp.float32),
            pltpu.SemaphoreType.DMA,
        ],
        mesh=plsc.VectorSubcoreMesh(core_axis_name="c", subcore_axis_name="s"),
    )


_BLK = 2048


def _dense_body(emb_ref, num_ref, w2a_ref, w2b_ref, b2_ref, wout_ref,
                bout_ref, x_ref, out_ref):
    acc = jnp.dot(emb_ref[...], w2a_ref[...], preferred_element_type=jnp.float32)
    acc = acc + jnp.dot(num_ref[...], w2b_ref[...], preferred_element_type=jnp.float32)
    acc = acc + b2_ref[...]
    h = acc * jax.nn.sigmoid(acc)
    x_ref[...] = h
    out_ref[...] = (
        jnp.dot(h, wout_ref[...], preferred_element_type=jnp.float32)
        + bout_ref[...]
    )


def _dense(embed, num, w2a, w2b, b2, wout, bout):
    grid = (B // _BLK,)
    return pl.pallas_call(
        _dense_body,
        grid=grid,
        in_specs=[
            pl.BlockSpec((_BLK, EMB_PAD), lambda i: (i, 0)),
            pl.BlockSpec((_BLK, NUM_F), lambda i: (i, 0)),
            pl.BlockSpec((EMB_PAD, H), lambda i: (0, 0)),
            pl.BlockSpec((NUM_F, H), lambda i: (0, 0)),
            pl.BlockSpec((1, H), lambda i: (0, 0)),
            pl.BlockSpec((H, 1), lambda i: (0, 0)),
            pl.BlockSpec((1, 1), lambda i: (0, 0)),
        ],
        out_specs=[
            pl.BlockSpec((_BLK, H), lambda i: (i, 0)),
            pl.BlockSpec((_BLK, 1), lambda i: (i, 0)),
        ],
        out_shape=[
            jax.ShapeDtypeStruct((B, H), jnp.float32),
            jax.ShapeDtypeStruct((B, 1), jnp.float32),
        ],
    )(embed, num, w2a, w2b, b2, wout, bout)


def kernel(cat, num, emb_table, W2, b2, Wout, bout):
    idx = cat.reshape(_NW * _CPW, _CHUNK).astype(jnp.int32)
    table_pad = jnp.pad(emb_table, ((0, 0), (0, EMB_PAD - EMB)))
    embed = _gather()(idx, table_pad)
    w2a_pad = jnp.pad(W2[:EMB], ((0, EMB_PAD - EMB), (0, 0)))
    x, out = _dense(
        embed,
        num,
        w2a_pad,
        W2[EMB:],
        b2.reshape(1, H),
        Wout,
        bout.reshape(1, 1),
    )
    return (x, out)


# speedup vs baseline: 1.2296x; 1.2296x over previous
import functools

import jax
import jax.numpy as jnp
from jax import lax
from jax.experimental import pallas as pl
from jax.experimental.pallas import tpu as pltpu
from jax.experimental.pallas import tpu_sc as plsc

VOCAB = 100000
EMB = 50
NUM_F = 128
B = 16384
H = 64

_NC = 2
_NS = 16
_NW = _NC * _NS
_BPW = B // _NW
_K = 16
_NCH = _BPW // _K


def _gather_body(idx_hbm, table_hbm, out_hbm, idx_v, rows_v, sem):
    wid = lax.axis_index("s") * _NC + lax.axis_index("c")
    pltpu.sync_copy(idx_hbm.at[pl.ds(wid, 1)], idx_v)

    def fire(ci):
        vec = idx_v[0, pl.ds(ci * _K, _K)]
        for j in range(_K):
            pltpu.async_copy(table_hbm.at[vec[j]], rows_v.at[ci * _K + j], sem)

    def drain(ci):
        for j in range(_K):
            i = ci * _K + j
            pltpu.make_async_copy(table_hbm.at[0], rows_v.at[i], sem).wait()

    fire(0)

    def chunk(ci, carry):
        fire(ci)
        drain(ci - 1)
        return carry

    lax.fori_loop(1, _NCH, chunk, 0, unroll=False)
    drain(_NCH - 1)
    pltpu.sync_copy(rows_v, out_hbm.at[pl.ds(wid * _BPW, _BPW)])


@functools.cache
def _gather():
    return pl.kernel(
        _gather_body,
        out_type=jax.ShapeDtypeStruct((B, EMB), jnp.float32),
        scratch_types=[
            pltpu.VMEM((1, _BPW), jnp.int32),
            pltpu.VMEM((_BPW, EMB), jnp.float32),
            pltpu.SemaphoreType.DMA,
        ],
        mesh=plsc.VectorSubcoreMesh(core_axis_name="c", subcore_axis_name="s"),
    )


_BLK = 2048


def _dense_body(emb_ref, num_ref, w2_ref, b2_ref, wout_ref, bout_ref,
                x_ref, out_ref):
    w2 = w2_ref[...]
    acc = jnp.dot(emb_ref[...], w2[0:EMB, :],
                  preferred_element_type=jnp.float32)
    acc = acc + jnp.dot(num_ref[...], w2[EMB:, :],
                        preferred_element_type=jnp.float32)
    acc = acc + b2_ref[...]
    h = acc * jax.nn.sigmoid(acc)
    x_ref[...] = h
    out_ref[...] = (
        jnp.dot(h, wout_ref[...], preferred_element_type=jnp.float32)
        + bout_ref[...]
    )


def _dense(embed, num, w2, b2, wout, bout):
    grid = (B // _BLK,)
    return pl.pallas_call(
        _dense_body,
        grid=grid,
        in_specs=[
            pl.BlockSpec((_BLK, EMB), lambda i: (i, 0)),
            pl.BlockSpec((_BLK, NUM_F), lambda i: (i, 0)),
            pl.BlockSpec((EMB + NUM_F, H), lambda i: (0, 0)),
            pl.BlockSpec((1, H), lambda i: (0, 0)),
            pl.BlockSpec((H, 1), lambda i: (0, 0)),
            pl.BlockSpec((1, 1), lambda i: (0, 0)),
        ],
        out_specs=[
            pl.BlockSpec((_BLK, H), lambda i: (i, 0)),
            pl.BlockSpec((_BLK, 1), lambda i: (i, 0)),
        ],
        out_shape=[
            jax.ShapeDtypeStruct((B, H), jnp.float32),
            jax.ShapeDtypeStruct((B, 1), jnp.float32),
        ],
    )(embed, num, w2, b2, wout, bout)


def kernel(cat, num, emb_table, W2, b2, Wout, bout):
    idx = cat.reshape(_NW, _BPW).astype(jnp.int32)
    embed = _gather()(idx, emb_table)
    x, out = _dense(
        embed,
        num,
        W2,
        b2.reshape(1, H),
        Wout,
        bout.reshape(1, 1),
    )
    return (x, out)


# speedup vs baseline: 2.1792x; 1.7723x over previous
import functools

import jax
import jax.numpy as jnp
from jax import lax
from jax.experimental import pallas as pl
from jax.experimental.pallas import tpu as pltpu
from jax.experimental.pallas import tpu_sc as plsc

VOCAB = 100000
EMB = 50
NUM_F = 128
B = 16384
H = 64

_NC = 2
_NS = 16
_NW = _NC * _NS
_HALF = B // 2
_UNROLL = 16


def _gather_body(idx_hbm, tableT_hbm, out_hbm, row_v, idx_v, out_v, sem):
    wid = lax.axis_index("s") * _NC + lax.axis_index("c")
    for p in range(2):
        f = p * _NW + wid

        @pl.when(f < EMB)
        def _():
            row_dma = pltpu.async_copy(tableT_hbm.at[f], row_v, sem)
            pltpu.sync_copy(idx_hbm.at[pl.ds(0, _HALF)], idx_v)
            row_dma.wait()
            for h in range(2):
                if h:
                    pltpu.sync_copy(idx_hbm.at[pl.ds(h * _HALF, _HALF)], idx_v)

                def gi(i, c):
                    for u in range(_UNROLL):
                        o = (i * _UNROLL + u) * 16
                        iv = idx_v[pl.ds(o, 16)]
                        out_v[pl.ds(o, 16)] = plsc.load_gather(row_v, [iv])
                    return c

                lax.fori_loop(0, _HALF // (16 * _UNROLL), gi, 0)
                pltpu.sync_copy(out_v, out_hbm.at[f, pl.ds(h * _HALF, _HALF)])


@functools.cache
def _gatherT():
    return pl.kernel(
        _gather_body,
        out_type=jax.ShapeDtypeStruct((EMB, B), jnp.float32),
        scratch_types=[
            pltpu.VMEM((VOCAB,), jnp.float32),
            pltpu.VMEM((_HALF,), jnp.int32),
            pltpu.VMEM((_HALF,), jnp.float32),
            pltpu.SemaphoreType.DMA,
        ],
        mesh=plsc.VectorSubcoreMesh(core_axis_name="c", subcore_axis_name="s"),
        compiler_params=pltpu.CompilerParams(needs_layout_passes=False),
    )


_BLK = 8192


def _partial_body(num_ref, w2t_ref, b2_ref, pT_ref):
    pT_ref[...] = b2_ref[...] + lax.dot_general(
        w2t_ref[:, EMB:], num_ref[...], (((1,), (1,)), ((), ())),
        preferred_element_type=jnp.float32)


def _partial(num, w2t, b2c):
    grid = (B // _BLK,)
    return pl.pallas_call(
        _partial_body,
        grid=grid,
        in_specs=[
            pl.BlockSpec((_BLK, NUM_F), lambda i: (i, 0)),
            pl.BlockSpec((H, EMB + NUM_F), lambda i: (0, 0)),
            pl.BlockSpec((H, 1), lambda i: (0, 0)),
        ],
        out_specs=pl.BlockSpec((H, _BLK), lambda i: (0, i)),
        out_shape=jax.ShapeDtypeStruct((H, B), jnp.float32),
    )(num, w2t, b2c)


def _final_body(embT_ref, pT_ref, w2t_ref, woutT_ref, bout_ref,
                xT_ref, outT_ref):
    accT = pT_ref[...] + jnp.dot(
        w2t_ref[:, :EMB], embT_ref[...], preferred_element_type=jnp.float32)
    hT = accT * jax.nn.sigmoid(accT)
    xT_ref[...] = hT
    outT_ref[...] = (
        jnp.dot(woutT_ref[...], hT, preferred_element_type=jnp.float32)
        + bout_ref[...]
    )


def _final(embT, pT, w2t, woutT, boutc):
    grid = (B // _BLK,)
    return pl.pallas_call(
        _final_body,
        grid=grid,
        in_specs=[
            pl.BlockSpec((EMB, _BLK), lambda i: (0, i)),
            pl.BlockSpec((H, _BLK), lambda i: (0, i)),
            pl.BlockSpec((H, EMB + NUM_F), lambda i: (0, 0)),
            pl.BlockSpec((1, H), lambda i: (0, 0)),
            pl.BlockSpec((1, 1), lambda i: (0, 0)),
        ],
        out_specs=[
            pl.BlockSpec((H, _BLK), lambda i: (0, i)),
            pl.BlockSpec((1, _BLK), lambda i: (0, i)),
        ],
        out_shape=[
            jax.ShapeDtypeStruct((H, B), jnp.float32),
            jax.ShapeDtypeStruct((1, B), jnp.float32),
        ],
    )(embT, pT, w2t, woutT, boutc)


def kernel(cat, num, emb_table, W2, b2, Wout, bout):
    idx = cat.reshape(B).astype(jnp.int32)
    embT = _gatherT()(idx, emb_table.T)
    w2t = W2.T
    pT = _partial(num, w2t, b2.reshape(H, 1))
    xT, outT = _final(embT, pT, w2t, Wout.T, bout.reshape(1, 1))
    return (xT.T, outT.T)


# speedup vs baseline: 2.2073x; 1.0129x over previous
import functools

import jax
import jax.numpy as jnp
from jax import lax
from jax.experimental import pallas as pl
from jax.experimental.pallas import tpu as pltpu
from jax.experimental.pallas import tpu_sc as plsc

VOCAB = 100000
EMB = 50
NUM_F = 128
B = 16384
H = 64

_NC = 2
_NS = 16
_NW = _NC * _NS
_HALF = B // 2
_UNROLL = 8


def _gather_body(idx_hbm, tableT_hbm, out_hbm, row_v, idx_v, out_v, sem):
    wid = lax.axis_index("s") * _NC + lax.axis_index("c")
    for p in range(2):
        f = p * _NW + wid

        @pl.when(f < EMB)
        def _():
            row_dma = pltpu.async_copy(tableT_hbm.at[f], row_v, sem)
            pltpu.sync_copy(idx_hbm.at[pl.ds(0, _HALF)], idx_v)
            row_dma.wait()
            for h in range(2):
                if h:
                    pltpu.sync_copy(idx_hbm.at[pl.ds(h * _HALF, _HALF)], idx_v)

                def gi(i, c):
                    for u in range(_UNROLL):
                        o = (i * _UNROLL + u) * 16
                        iv = idx_v[pl.ds(o, 16)]
                        out_v[pl.ds(o, 16)] = plsc.load_gather(row_v, [iv])
                    return c

                lax.fori_loop(0, _HALF // (16 * _UNROLL), gi, 0)
                pltpu.sync_copy(out_v, out_hbm.at[f, pl.ds(h * _HALF, _HALF)])


@functools.cache
def _gatherT():
    return pl.kernel(
        _gather_body,
        out_type=jax.ShapeDtypeStruct((EMB, B), jnp.float32),
        scratch_types=[
            pltpu.VMEM((VOCAB,), jnp.float32),
            pltpu.VMEM((_HALF,), jnp.int32),
            pltpu.VMEM((_HALF,), jnp.float32),
            pltpu.SemaphoreType.DMA,
        ],
        mesh=plsc.VectorSubcoreMesh(core_axis_name="c", subcore_axis_name="s"),
        compiler_params=pltpu.CompilerParams(needs_layout_passes=False),
    )


_BLK = 8192


def _partial_body(num_ref, w2t_ref, b2_ref, pT_ref):
    pT_ref[...] = b2_ref[...] + lax.dot_general(
        w2t_ref[:, EMB:], num_ref[...], (((1,), (1,)), ((), ())),
        preferred_element_type=jnp.float32)


def _partial(num, w2t, b2c):
    grid = (B // _BLK,)
    return pl.pallas_call(
        _partial_body,
        grid=grid,
        in_specs=[
            pl.BlockSpec((_BLK, NUM_F), lambda i: (i, 0)),
            pl.BlockSpec((H, EMB + NUM_F), lambda i: (0, 0)),
            pl.BlockSpec((H, 1), lambda i: (0, 0)),
        ],
        out_specs=pl.BlockSpec((H, _BLK), lambda i: (0, i)),
        out_shape=jax.ShapeDtypeStruct((H, B), jnp.float32),
    )(num, w2t, b2c)


def _final_body(embT_ref, pT_ref, w2t_ref, woutT_ref, bout_ref,
                xT_ref, outT_ref):
    accT = pT_ref[...] + jnp.dot(
        w2t_ref[:, :EMB], embT_ref[...], preferred_element_type=jnp.float32)
    hT = accT * jax.nn.sigmoid(accT)
    xT_ref[...] = hT
    outT_ref[...] = (
        jnp.dot(woutT_ref[...], hT, preferred_element_type=jnp.float32)
        + bout_ref[...]
    )


def _final(embT, pT, w2t, woutT, boutc):
    grid = (B // _BLK,)
    return pl.pallas_call(
        _final_body,
        grid=grid,
        in_specs=[
            pl.BlockSpec((EMB, _BLK), lambda i: (0, i)),
            pl.BlockSpec((H, _BLK), lambda i: (0, i)),
            pl.BlockSpec((H, EMB + NUM_F), lambda i: (0, 0)),
            pl.BlockSpec((1, H), lambda i: (0, 0)),
            pl.BlockSpec((1, 1), lambda i: (0, 0)),
        ],
        out_specs=[
            pl.BlockSpec((H, _BLK), lambda i: (0, i)),
            pl.BlockSpec((1, _BLK), lambda i: (0, i)),
        ],
        out_shape=[
            jax.ShapeDtypeStruct((H, B), jnp.float32),
            jax.ShapeDtypeStruct((1, B), jnp.float32),
        ],
    )(embT, pT, w2t, woutT, boutc)


def kernel(cat, num, emb_table, W2, b2, Wout, bout):
    idx = cat.reshape(B).astype(jnp.int32)
    embT = _gatherT()(idx, emb_table.T)
    w2t = W2.T
    pT = _partial(num, w2t, b2.reshape(H, 1))
    xT, outT = _final(embT, pT, w2t, Wout.T, bout.reshape(1, 1))
    return (xT.T, outT.T)
